# Initial kernel scaffold; baseline (speedup 1.0000x reference)
import jax
import jax.numpy as jnp
from jax.experimental import pallas as pl


def kernel(x, spd, sp_enc):
    raise NotImplementedError("write your pallas kernel here")


# speedup vs baseline: 55.2361x; 55.2361x over previous
import functools

import jax
import jax.numpy as jnp
from jax import lax
from jax.experimental import pallas as pl
from jax.experimental.pallas import tpu as pltpu
from jax.experimental.pallas import tpu_sc as plsc

_L = 16
_NW = 32
_NP = 512
_CHUNK = (_NP * _NP) // _NW
_HNO = 8


def _sc_gather_plane(spd_flat, tbl_t):
    mesh = plsc.VectorSubcoreMesh(core_axis_name="c", subcore_axis_name="s")

    @functools.partial(
        pl.kernel,
        mesh=mesh,
        compiler_params=pltpu.CompilerParams(needs_layout_passes=False),
        out_type=jax.ShapeDtypeStruct((_HNO, _NP * _NP), jnp.float32),
        scratch_types=[
            pltpu.VMEM((_CHUNK,), jnp.int32),
            pltpu.VMEM((_HNO * 64,), jnp.float32),
            pltpu.VMEM((_HNO, _CHUNK), jnp.float32),
        ],
    )
    def run(spd_hbm, tbl_hbm, out_hbm, idx_v, tbl_v, acc_v):
        wid = lax.axis_index("s") * 2 + lax.axis_index("c")
        base = wid * _CHUNK
        pltpu.sync_copy(spd_hbm.at[pl.ds(base, _CHUNK)], idx_v)
        pltpu.sync_copy(tbl_hbm, tbl_v)

        def body(k, carry):
            idx_vec = idx_v[pl.ds(k * _L, _L)]
            for h in range(_HNO):
                vals = plsc.load_gather(tbl_v, [idx_vec + (h * 64)])
                acc_v[h, pl.ds(k * _L, _L)] = vals
            return carry

        lax.fori_loop(0, _CHUNK // _L, body, 0)
        for h in range(_HNO):
            pltpu.sync_copy(acc_v.at[h], out_hbm.at[h, pl.ds(base, _CHUNK)])

    return run(spd_flat, tbl_t)


def _tc_body(plane_ref, out_ref):
    out_ref[0, 0] = plane_ref[0, : 501, : 501]


def _tc_broadcast(plane, B):
    return pl.pallas_call(
        _tc_body,
        grid=(_HNO, B),
        in_specs=[pl.BlockSpec((1, _NP, _NP), lambda h, b: (h, 0, 0))],
        out_specs=pl.BlockSpec((1, 1, 501, 501), lambda h, b: (b, h, 0, 0)),
        out_shape=jax.ShapeDtypeStruct((B, _HNO, 501, 501), jnp.float32),
    )(plane)


def kernel(x, spd, sp_enc):
    B = x.shape[0]
    N = x.shape[2]
    table = sp_enc.at[0].set(0.0)
    tbl_t = jnp.zeros((_HNO, 64), jnp.float32).at[:, : 51].set(table.T)
    tbl_t = tbl_t.reshape(-1)
    spd_b = (
        jnp.zeros((_NP, _NP), jnp.int32)
        .at[1 : N + 1, 1 : N + 1]
        .set(spd.astype(jnp.int32))
    )
    plane = _sc_gather_plane(spd_b.reshape(-1), tbl_t)
    return _tc_broadcast(plane.reshape(_HNO, _NP, _NP), B)


# speedup vs baseline: 64.9608x; 1.1761x over previous
import functools

import jax
import jax.numpy as jnp
from jax import lax
from jax.experimental import pallas as pl
from jax.experimental.pallas import tpu as pltpu
from jax.experimental.pallas import tpu_sc as plsc

_L = 16
_NW = 32
_NP = 512
_CHUNK = (_NP * _NP) // _NW
_HNO = 8


def _sc_gather_plane(spd_flat, tbl_t):
    mesh = plsc.VectorSubcoreMesh(core_axis_name="c", subcore_axis_name="s")

    @functools.partial(
        pl.kernel,
        mesh=mesh,
        compiler_params=pltpu.CompilerParams(needs_layout_passes=False),
        out_type=jax.ShapeDtypeStruct((_HNO, _NP * _NP), jnp.float32),
        scratch_types=[
            pltpu.VMEM((_CHUNK,), jnp.int32),
            pltpu.VMEM((_HNO * 64,), jnp.float32),
            pltpu.VMEM((_HNO, _CHUNK), jnp.float32),
        ],
    )
    def run(spd_hbm, tbl_hbm, out_hbm, idx_v, tbl_v, acc_v):
        wid = lax.axis_index("s") * 2 + lax.axis_index("c")
        base = wid * _CHUNK
        pltpu.sync_copy(spd_hbm.at[pl.ds(base, _CHUNK)], idx_v)
        pltpu.sync_copy(tbl_hbm, tbl_v)

        def body(k, carry):
            idx_vec = idx_v[pl.ds(k * _L, _L)]
            for h in range(_HNO):
                vals = plsc.load_gather(tbl_v, [idx_vec + (h * 64)])
                acc_v[h, pl.ds(k * _L, _L)] = vals
            return carry

        lax.fori_loop(0, _CHUNK // _L, body, 0)
        for h in range(_HNO):
            pltpu.sync_copy(acc_v.at[h], out_hbm.at[h, pl.ds(base, _CHUNK)])

    return run(spd_flat, tbl_t)


def _tc_body(plane_ref, out_ref):
    out_ref[0] = plane_ref[:, : 501, : 501]


def _tc_broadcast(plane, B):
    return pl.pallas_call(
        _tc_body,
        grid=(B,),
        in_specs=[pl.BlockSpec((_HNO, _NP, _NP), lambda b: (0, 0, 0))],
        out_specs=pl.BlockSpec((1, _HNO, 501, 501), lambda b: (b, 0, 0, 0)),
        out_shape=jax.ShapeDtypeStruct((B, _HNO, 501, 501), jnp.float32),
    )(plane)


def kernel(x, spd, sp_enc):
    B = x.shape[0]
    N = x.shape[2]
    table = sp_enc.at[0].set(0.0)
    tbl_t = jnp.zeros((_HNO, 64), jnp.float32).at[:, : 51].set(table.T)
    tbl_t = tbl_t.reshape(-1)
    spd_b = (
        jnp.zeros((_NP, _NP), jnp.int32)
        .at[1 : N + 1, 1 : N + 1]
        .set(spd.astype(jnp.int32))
    )
    plane = _sc_gather_plane(spd_b.reshape(-1), tbl_t)
    return _tc_broadcast(plane.reshape(_HNO, _NP, _NP), B)
